# baseline (speedup 1.0000x reference)
import jax
import jax.numpy as jnp
from jax.experimental import pallas as pl

N_NODES = 100000
N_SPECIES = 64
BLOCK = 2000


def _onehot_body(idx_ref, out_ref):
    t = idx_ref[...]
    cols = jax.lax.broadcasted_iota(jnp.int32, (BLOCK, N_SPECIES), 1)
    out_ref[...] = (t == cols).astype(jnp.float32)


def kernel(atom_types):
    grid = (N_NODES // BLOCK,)
    return pl.pallas_call(
        _onehot_body,
        grid=grid,
        in_specs=[pl.BlockSpec((BLOCK, 1), lambda i: (i, 0))],
        out_specs=pl.BlockSpec((BLOCK, N_SPECIES), lambda i: (i, 0)),
        out_shape=jax.ShapeDtypeStruct((N_NODES, N_SPECIES), jnp.float32),
    )(atom_types)


# speedup vs baseline: 1.1249x; 1.1249x over previous
import functools
import jax
import jax.numpy as jnp
from jax import lax
from jax.experimental import pallas as pl
from jax.experimental.pallas import tpu as pltpu
from jax.experimental.pallas import tpu_sc as plsc

N_NODES = 100000
N_SPECIES = 64
NW = 32
CH_ROWS = 400
CH_GROUPS = CH_ROWS // 16
CH_WORDS = CH_ROWS * N_SPECIES
N_CHUNKS = N_NODES // CH_ROWS
MAX_T = (N_CHUNKS + NW - 1) // NW


def _sc_body(idx_hbm, out_hbm, buf0, buf1, idx0, idx1, sem0, sem1):
    wid = lax.axis_index("s") * 2 + lax.axis_index("c")
    riota = lax.iota(jnp.int32, 16)
    riota64 = riota * N_SPECIES
    ones = jnp.full((16,), 1.0, jnp.float32)
    zeros = jnp.full((16,), 0.0, jnp.float32)

    bufs = (buf0, buf1)
    idxs = (idx0, idx1)
    sems = (sem0, sem1)

    def _zi(i, _):
        offs = riota + i * 16
        plsc.store_scatter(buf0, [offs], zeros)
        plsc.store_scatter(buf1, [offs], zeros)
        return 0
    lax.fori_loop(0, CH_WORDS // 16, _zi, 0)

    for t in range(MAX_T):
        buf, idxb, sem = bufs[t % 2], idxs[t % 2], sems[t % 2]
        ci = wid + t * NW

        @pl.when(ci < N_CHUNKS)
        def _chunk():
            if t >= 2:
                pltpu.make_async_copy(
                    buf, out_hbm.at[pl.ds((ci - 2 * NW) * CH_WORDS, CH_WORDS)], sem
                ).wait()
                for j in range(CH_GROUPS):
                    iv = idxb[pl.ds(j * 16, 16)]
                    plsc.store_scatter(buf, [riota64 + (j * 1024) + iv], zeros)
            pltpu.sync_copy(idx_hbm.at[pl.ds(ci * CH_ROWS, CH_ROWS)], idxb)
            for j in range(CH_GROUPS):
                iv = idxb[pl.ds(j * 16, 16)]
                plsc.store_scatter(buf, [riota64 + (j * 1024) + iv], ones)
            pltpu.async_copy(buf, out_hbm.at[pl.ds(ci * CH_WORDS, CH_WORDS)], sem)

    pltpu.make_async_copy(buf0, out_hbm.at[pl.ds(0, CH_WORDS)], sem0).wait()
    pltpu.make_async_copy(buf1, out_hbm.at[pl.ds(0, CH_WORDS)], sem1).wait()


@jax.jit
def kernel(atom_types):
    idx = atom_types.reshape(N_NODES)
    mesh = plsc.VectorSubcoreMesh(core_axis_name="c", subcore_axis_name="s")
    out_flat = pl.kernel(
        _sc_body,
        out_type=jax.ShapeDtypeStruct((N_NODES * N_SPECIES,), jnp.float32),
        mesh=mesh,
        compiler_params=pltpu.CompilerParams(needs_layout_passes=False),
        scratch_types=[
            pltpu.VMEM((CH_WORDS,), jnp.float32),
            pltpu.VMEM((CH_WORDS,), jnp.float32),
            pltpu.VMEM((CH_ROWS,), jnp.int32),
            pltpu.VMEM((CH_ROWS,), jnp.int32),
            pltpu.SemaphoreType.DMA,
            pltpu.SemaphoreType.DMA,
        ],
    )(idx)
    return out_flat.reshape(N_NODES, N_SPECIES)


# speedup vs baseline: 1.5091x; 1.3415x over previous
import jax
import jax.numpy as jnp
from jax import lax
from jax.experimental import pallas as pl
from jax.experimental.pallas import tpu as pltpu
from jax.experimental.pallas import tpu_sc as plsc

N_NODES = 100000
N_SPECIES = 64
NW = 32
CH_ROWS = 400
CH_GROUPS = CH_ROWS // 16
N_CHUNKS = N_NODES // CH_ROWS
MAX_T = (N_CHUNKS + NW - 1) // NW


def _sc_body(idx_hbm, out_hbm, buf0, buf1, idx0, idx1, sem0, sem1):
    wid = lax.axis_index("s") * 2 + lax.axis_index("c")
    riota = lax.iota(jnp.int32, 16)
    ones = jnp.full((16,), 1.0, jnp.float32)
    zeros = jnp.full((16,), 0.0, jnp.float32)

    bufs = (buf0, buf1)
    idxs = (idx0, idx1)
    sems = (sem0, sem1)

    def _zi(r, _):
        for k in range(N_SPECIES // 16):
            buf0[r, pl.ds(k * 16, 16)] = zeros
            buf1[r, pl.ds(k * 16, 16)] = zeros
        return 0
    lax.fori_loop(0, CH_ROWS, _zi, 0)

    for t in range(MAX_T):
        buf, idxb, sem = bufs[t % 2], idxs[t % 2], sems[t % 2]
        ci = wid + t * NW

        @pl.when(ci < N_CHUNKS)
        def _chunk():
            if t >= 2:
                pltpu.make_async_copy(
                    buf,
                    out_hbm.at[pl.ds((ci - 2 * NW) * CH_ROWS, CH_ROWS), :],
                    sem,
                ).wait()
                for j in range(CH_GROUPS):
                    iv = idxb[pl.ds(j * 16, 16)]
                    plsc.store_scatter(buf, [riota + (j * 16), iv], zeros)
            pltpu.sync_copy(idx_hbm.at[pl.ds(ci * CH_ROWS, CH_ROWS)], idxb)
            for j in range(CH_GROUPS):
                iv = idxb[pl.ds(j * 16, 16)]
                plsc.store_scatter(buf, [riota + (j * 16), iv], ones)
            pltpu.async_copy(
                buf, out_hbm.at[pl.ds(ci * CH_ROWS, CH_ROWS), :], sem
            )

    pltpu.make_async_copy(buf0, out_hbm.at[pl.ds(0, CH_ROWS), :], sem0).wait()
    pltpu.make_async_copy(buf1, out_hbm.at[pl.ds(0, CH_ROWS), :], sem1).wait()


@jax.jit
def kernel(atom_types):
    idx = atom_types.reshape(N_NODES)
    mesh = plsc.VectorSubcoreMesh(core_axis_name="c", subcore_axis_name="s")
    return pl.kernel(
        _sc_body,
        out_type=jax.ShapeDtypeStruct((N_NODES, N_SPECIES), jnp.float32),
        mesh=mesh,
        compiler_params=pltpu.CompilerParams(needs_layout_passes=False),
        scratch_types=[
            pltpu.VMEM((CH_ROWS, N_SPECIES), jnp.float32),
            pltpu.VMEM((CH_ROWS, N_SPECIES), jnp.float32),
            pltpu.VMEM((CH_ROWS,), jnp.int32),
            pltpu.VMEM((CH_ROWS,), jnp.int32),
            pltpu.SemaphoreType.DMA,
            pltpu.SemaphoreType.DMA,
        ],
    )(idx)
